# baseline (speedup 1.0000x reference)
import jax
import jax.numpy as jnp
from jax.experimental import pallas as pl
from jax.experimental.pallas import tpu as pltpu

Q_LEN = 2048
K_LEN = 2048
EMB = 64
NSEM = 8


def _body(rev_ref, out_ref, sems):
    def dma(i):
        return pltpu.make_async_copy(
            rev_ref.at[pl.ds(K_LEN - 1 - i, K_LEN), :],
            out_ref.at[i],
            sems.at[i % NSEM])

    def loop(i, carry):
        @pl.when(i >= NSEM)
        def _():
            dma(i - NSEM).wait()
        dma(i).start()
        return carry

    jax.lax.fori_loop(0, Q_LEN, loop, 0)

    def tail(i, carry):
        dma(i).wait()
        return carry

    jax.lax.fori_loop(Q_LEN - NSEM, Q_LEN, tail, 0)


def kernel(q, k, rel_emb):
    rev = jnp.flip(rel_emb, axis=0)
    out = pl.pallas_call(
        _body,
        in_specs=[
            pl.BlockSpec(memory_space=pltpu.VMEM),
        ],
        out_specs=pl.BlockSpec(memory_space=pl.ANY),
        out_shape=jax.ShapeDtypeStruct((Q_LEN, K_LEN, EMB), jnp.float32),
        scratch_shapes=[pltpu.SemaphoreType.DMA((NSEM,))],
    )(rev)
    return out


# speedup vs baseline: 1.0327x; 1.0327x over previous
import jax
import jax.numpy as jnp
from jax.experimental import pallas as pl
from jax.experimental.pallas import tpu as pltpu

Q_LEN = 2048
K_LEN = 2048
EMB = 64
ROW128 = K_LEN * EMB // 128
PAIRS_IN_FLIGHT = 4


def _body(reva_ref, revb_ref, out_ref, sems):
    def dma_even(p):
        return pltpu.make_async_copy(
            revb_ref.at[pl.ds(ROW128 - 1 - p, ROW128), :],
            out_ref.at[2 * p],
            sems.at[(2 * p) % (2 * PAIRS_IN_FLIGHT)])

    def dma_odd(p):
        return pltpu.make_async_copy(
            reva_ref.at[pl.ds(ROW128 - 1 - p, ROW128), :],
            out_ref.at[2 * p + 1],
            sems.at[(2 * p + 1) % (2 * PAIRS_IN_FLIGHT)])

    def loop(p, carry):
        @pl.when(p >= PAIRS_IN_FLIGHT)
        def _():
            dma_even(p - PAIRS_IN_FLIGHT).wait()
            dma_odd(p - PAIRS_IN_FLIGHT).wait()
        dma_even(p).start()
        dma_odd(p).start()
        return carry

    jax.lax.fori_loop(0, Q_LEN // 2, loop, 0)

    def tail(p, carry):
        dma_even(p).wait()
        dma_odd(p).wait()
        return carry

    jax.lax.fori_loop(Q_LEN // 2 - PAIRS_IN_FLIGHT, Q_LEN // 2, tail, 0)


def kernel(q, k, rel_emb):
    rev_flat = jnp.flip(rel_emb, axis=0).reshape(-1)
    reva = rev_flat.reshape(2 * K_LEN * EMB // 128, 128)
    revb = jax.lax.dynamic_slice(rev_flat, (64,),
                                 ((2 * ROW128 - 1) * 128,)).reshape(
                                     2 * ROW128 - 1, 128)
    out = pl.pallas_call(
        _body,
        in_specs=[
            pl.BlockSpec(memory_space=pltpu.VMEM),
            pl.BlockSpec(memory_space=pltpu.VMEM),
        ],
        out_specs=pl.BlockSpec(memory_space=pl.ANY),
        out_shape=jax.ShapeDtypeStruct((Q_LEN, ROW128, 128), jnp.float32),
        scratch_shapes=[pltpu.SemaphoreType.DMA((2 * PAIRS_IN_FLIGHT,))],
    )(reva, revb)
    return out.reshape(Q_LEN, K_LEN, EMB)
